# Initial kernel scaffold; baseline (speedup 1.0000x reference)
import jax
import jax.numpy as jnp
from jax.experimental import pallas as pl


def kernel(inputs, embedding):
    raise NotImplementedError("write your pallas kernel here")


# speedup vs baseline: 2.7577x; 2.7577x over previous
import functools

import jax
import jax.numpy as jnp
from jax import lax
from jax.experimental import pallas as pl
from jax.experimental.pallas import tpu as pltpu
from jax.experimental.pallas import tpu_sc as plsc

_D = 128
_CH = 128


@functools.lru_cache(maxsize=None)
def _make_gather(B, V):
    info = plsc.get_sparse_core_info()
    nw = info.num_cores * info.num_subcores
    assert B % (nw * _CH) == 0
    b_per_w = B // nw
    n_ch = b_per_w // _CH
    mesh = plsc.VectorSubcoreMesh(core_axis_name="c", subcore_axis_name="s")

    @functools.partial(
        pl.kernel,
        mesh=mesh,
        out_type=jax.ShapeDtypeStruct((B, _D), jnp.float32),
        scratch_types=[
            pltpu.VMEM((_CH,), jnp.int32),
            pltpu.VMEM((_CH, _D), jnp.float32),
            pltpu.SemaphoreType.DMA,
        ],
    )
    def gather_kernel(idx_hbm, table_hbm, out_hbm, idx_v, rows_v, sem):
        wid = lax.axis_index("s") * info.num_cores + lax.axis_index("c")
        base = wid * b_per_w

        def body(g, carry):
            off = base + g * _CH
            pltpu.sync_copy(idx_hbm.at[pl.ds(off, _CH)], idx_v)
            pltpu.async_copy(table_hbm.at[idx_v], rows_v, sem).wait()
            pltpu.sync_copy(rows_v, out_hbm.at[pl.ds(off, _CH)])
            return carry

        lax.fori_loop(0, n_ch, body, 0)

    return gather_kernel


def kernel(inputs, embedding):
    batch, steps = inputs.shape
    vocab, d = embedding.shape
    assert d == _D
    flat_idx = inputs.reshape(-1).astype(jnp.int32)
    out = _make_gather(batch * steps, vocab)(flat_idx, embedding)
    return out.reshape(batch, steps, d)


# speedup vs baseline: 3.3274x; 1.2066x over previous
import functools

import jax
import jax.numpy as jnp
from jax import lax
from jax.experimental import pallas as pl
from jax.experimental.pallas import tpu as pltpu
from jax.experimental.pallas import tpu_sc as plsc

_D = 128
_CH = 128


@functools.lru_cache(maxsize=None)
def _make_gather(B, V):
    info = plsc.get_sparse_core_info()
    nw = info.num_cores * info.num_subcores
    assert B % (nw * _CH) == 0
    b_per_w = B // nw
    n_ch = b_per_w // _CH
    assert n_ch % 2 == 0 and n_ch >= 4
    mesh = plsc.VectorSubcoreMesh(core_axis_name="c", subcore_axis_name="s")

    @functools.partial(
        pl.kernel,
        mesh=mesh,
        out_type=jax.ShapeDtypeStruct((B, _D), jnp.float32),
        scratch_types=[
            pltpu.VMEM((n_ch, _CH), jnp.int32),
            pltpu.VMEM((2, _CH, _D), jnp.float32),
            pltpu.SemaphoreType.DMA((2,)),
            pltpu.SemaphoreType.DMA((2,)),
        ],
    )
    def gather_kernel(idx_hbm, table_hbm, out_hbm, idx_v, rows_v, gsem, osem):
        wid = lax.axis_index("s") * info.num_cores + lax.axis_index("c")
        base = wid * b_per_w
        pltpu.sync_copy(idx_hbm.at[wid], idx_v)

        def gather_start(g, b):
            pltpu.async_copy(table_hbm.at[idx_v.at[g]], rows_v.at[b], gsem.at[b])

        def gather_wait(b):
            pltpu.make_async_copy(
                table_hbm.at[idx_v.at[0]], rows_v.at[b], gsem.at[b]
            ).wait()

        def out_start(g, b):
            pltpu.async_copy(
                rows_v.at[b], out_hbm.at[pl.ds(base + g * _CH, _CH)], osem.at[b]
            )

        def out_wait(b):
            pltpu.make_async_copy(
                rows_v.at[b], out_hbm.at[pl.ds(base, _CH)], osem.at[b]
            ).wait()

        gather_start(0, 0)
        gather_start(1, 1)
        gather_wait(0)
        out_start(0, 0)

        def body(t, carry):
            for i in range(2):
                g = 2 * t + 1 + i
                b = (1 + i) % 2
                nb = 1 - b
                out_wait(nb)
                gather_start(g + 1, nb)
                gather_wait(b)
                out_start(g, b)
            return carry

        lax.fori_loop(0, (n_ch - 2) // 2, body, 0)

        gather_wait(1)
        out_start(n_ch - 1, 1)
        out_wait(0)
        out_wait(1)

    return gather_kernel


def kernel(inputs, embedding):
    batch, steps = inputs.shape
    vocab, d = embedding.shape
    assert d == _D
    B = batch * steps
    info = plsc.get_sparse_core_info()
    nw = info.num_cores * info.num_subcores
    n_ch = B // (nw * _CH)
    flat_idx = inputs.reshape(nw, n_ch, _CH).astype(jnp.int32)
    out = _make_gather(B, vocab)(flat_idx, embedding)
    return out.reshape(batch, steps, d)
